# baseline (speedup 1.0000x reference)
import jax
import jax.numpy as jnp
from jax import lax
from jax.experimental import pallas as pl
from jax.experimental.pallas import tpu as pltpu

_EPS = 1e-06
_RATIO = 3.0
_B = 8
_GRID = 4
_H = 512
_W = 512
_ROWS = _B * _H
_CHUNK = 4
_CR = _ROWS // _CHUNK


def _ohem_body(cs0_ref, cs1_ref, cs2_ref, cs3_ref,
               lb0_ref, lb1_ref, lb2_ref, lb3_ref,
               mk0_ref, mk1_ref, mk2_ref, mk3_ref,
               out_ref, bits_ref, acc_ref):
    i = pl.program_id(0)

    @pl.when(i == 0)
    def _init():
        acc_ref[0] = 0.0
        acc_ref[1] = 0.0
        acc_ref[2] = 0.0
        acc_ref[3] = 1e30
        acc_ref[4] = 0.0
        acc_ref[5] = 0.0

    rows = _B // _GRID * (_H // 4)
    for h, (cref, lref, mref) in enumerate(
            [(cs0_ref, lb0_ref, mk0_ref), (cs1_ref, lb1_ref, mk1_ref),
             (cs2_ref, lb2_ref, mk2_ref), (cs3_ref, lb3_ref, mk3_ref)]):
        cs = cref[...].reshape(rows, _W)
        lb = lref[...].reshape(rows, _W)
        mk = mref[...].reshape(rows, _W)
        p = jnp.clip(cs, 1e-12, 1.0 - 1e-12)
        posm = lb * mk
        negm = (1.0 - lb) * mk
        acc_ref[0] += jnp.sum(posm)
        acc_ref[1] += jnp.sum(negm)
        acc_ref[2] += jnp.sum(jnp.where(posm > 0.0, -jnp.log(p), 0.0))
        nl = jnp.where(negm > 0.0, -jnp.log1p(-p), 0.0)
        acc_ref[5] += jnp.sum(nl)
        acc_ref[3] = jnp.minimum(acc_ref[3],
                                 jnp.min(jnp.where(negm > 0.0, nl, 1e30)))
        acc_ref[4] = jnp.maximum(acc_ref[4], jnp.max(nl))
        bits_ref[pl.ds((4 * i + h) * rows, rows), :] = nl

    @pl.when(i == pl.num_programs(0) - 1)
    def _select():
        pos_sum = acc_ref[0]
        neg_sum = acc_ref[1]
        pos_loss_sum = acc_ref[2]
        pos_cnt = jnp.floor(pos_sum)
        k = jnp.floor(jnp.minimum(neg_sum, pos_sum * _RATIO))

        def all_negs():
            return acc_ref[5]

        def search():
            def counts_ge(t1, t2, t3):
                def blk(j, c):
                    x = bits_ref[pl.ds(j * _CR, _CR), :]
                    xb = lax.bitcast_convert_type(x, jnp.int32)
                    return (c[0] + jnp.sum(jnp.where(xb >= t1, 1.0, 0.0)),
                            c[1] + jnp.sum(jnp.where(xb >= t2, 1.0, 0.0)),
                            c[2] + jnp.sum(jnp.where(xb >= t3, 1.0, 0.0)))
                return lax.fori_loop(0, _CHUNK, blk, (0.0, 0.0, 0.0))

            def quad(lohi):
                lo, hi = lohi
                d = (hi - lo + 3) // 4
                t1 = lo + d
                t2 = t1 + d
                t3 = t2 + d
                c1, c2, c3 = counts_ge(t1, t2, t3)
                lo = jnp.where(c3 >= k, t3,
                               jnp.where(c2 >= k, t2,
                                         jnp.where(c1 >= k, t1, lo)))
                hi = jnp.where(c1 < k, t1,
                               jnp.where(c2 < k, t2,
                                         jnp.where(c3 < k, t3, hi)))
                return (lo, hi)

            lo0 = jnp.maximum(
                lax.bitcast_convert_type(acc_ref[3], jnp.int32),
                jnp.int32(1))
            hi0 = jnp.maximum(
                lax.bitcast_convert_type(acc_ref[4], jnp.int32) + 1,
                lo0 + 1)
            v, _hi = lax.while_loop(lambda lh: lh[1] - lh[0] > 1, quad,
                                    (lo0, hi0))
            lossv = lax.bitcast_convert_type(v, jnp.float32)

            def blk2(j, carry):
                cgt, sgt = carry
                x = bits_ref[pl.ds(j * _CR, _CR), :]
                xb = lax.bitcast_convert_type(x, jnp.int32)
                gt = xb > v
                cgt += jnp.sum(jnp.where(gt, 1.0, 0.0))
                sgt += jnp.sum(jnp.where(gt, x, 0.0))
                return (cgt, sgt)

            cgt, sgt = lax.fori_loop(0, _CHUNK, blk2, (0.0, 0.0))
            return sgt + jnp.where(k > cgt, (k - cgt) * lossv, 0.0)

        top_neg = lax.cond(k >= neg_sum, all_negs, search)
        out_ref[0, 0] = (pos_loss_sum + top_neg) / (pos_cnt + k + _EPS)


def kernel(cls_score, label, mask):
    out = pl.pallas_call(
        _ohem_body,
        grid=(_GRID,),
        in_specs=[
            pl.BlockSpec((_B // _GRID, _H // 4, _W), lambda i: (i, 0, 0)),
            pl.BlockSpec((_B // _GRID, _H // 4, _W), lambda i: (i, 1, 0)),
            pl.BlockSpec((_B // _GRID, _H // 4, _W), lambda i: (i, 2, 0)),
            pl.BlockSpec((_B // _GRID, _H // 4, _W), lambda i: (i, 3, 0)),
            pl.BlockSpec((_B // _GRID, _H // 4, _W), lambda i: (i, 0, 0)),
            pl.BlockSpec((_B // _GRID, _H // 4, _W), lambda i: (i, 1, 0)),
            pl.BlockSpec((_B // _GRID, _H // 4, _W), lambda i: (i, 2, 0)),
            pl.BlockSpec((_B // _GRID, _H // 4, _W), lambda i: (i, 3, 0)),
            pl.BlockSpec((_B // _GRID, _H // 4, _W), lambda i: (i, 0, 0)),
            pl.BlockSpec((_B // _GRID, _H // 4, _W), lambda i: (i, 1, 0)),
            pl.BlockSpec((_B // _GRID, _H // 4, _W), lambda i: (i, 2, 0)),
            pl.BlockSpec((_B // _GRID, _H // 4, _W), lambda i: (i, 3, 0)),
        ],
        out_specs=pl.BlockSpec(memory_space=pltpu.SMEM),
        out_shape=jax.ShapeDtypeStruct((1, 1), jnp.float32),
        scratch_shapes=[
            pltpu.VMEM((_ROWS, _W), jnp.float32),
            pltpu.SMEM((6,), jnp.float32),
        ],
        compiler_params=pltpu.CompilerParams(
            dimension_semantics=("arbitrary",),
        ),
    )(cls_score, cls_score, cls_score, cls_score,
      label, label, label, label, mask, mask, mask, mask)
    return out.reshape(())


# speedup vs baseline: 1.0307x; 1.0307x over previous
import jax
import jax.numpy as jnp
from jax import lax
from jax.experimental import pallas as pl
from jax.experimental.pallas import tpu as pltpu

_EPS = 1e-06
_RATIO = 3.0
_B = 8
_GRID = 4
_H = 512
_W = 512
_ROWS = _B * _H
_CHUNK = 4
_CR = _ROWS // _CHUNK


def _ohem_body(cs0_ref, cs1_ref, lb0_ref, lb1_ref, mk0_ref, mk1_ref,
               out_ref, bits_ref, acc_ref):
    i = pl.program_id(0)

    @pl.when(i == 0)
    def _init():
        acc_ref[0] = 0.0
        acc_ref[1] = 0.0
        acc_ref[2] = 0.0
        acc_ref[3] = 1e30
        acc_ref[4] = 0.0
        acc_ref[5] = 0.0

    hw = _W // 2
    rows = _B // _GRID * _H
    for h, (cref, lref, mref) in enumerate(
            [(cs0_ref, lb0_ref, mk0_ref), (cs1_ref, lb1_ref, mk1_ref)]):
        cs = cref[...].reshape(rows, hw)
        lb = lref[...].reshape(rows, hw)
        mk = mref[...].reshape(rows, hw)
        p = jnp.clip(cs, 1e-12, 1.0 - 1e-12)
        posm = lb * mk
        negm = (1.0 - lb) * mk
        acc_ref[0] += jnp.sum(posm)
        acc_ref[1] += jnp.sum(negm)
        acc_ref[2] += jnp.sum(jnp.where(posm > 0.0, -jnp.log(p), 0.0))
        nl = jnp.where(negm > 0.0, -jnp.log1p(-p), 0.0)
        acc_ref[5] += jnp.sum(nl)
        acc_ref[3] = jnp.minimum(acc_ref[3],
                                 jnp.min(jnp.where(negm > 0.0, nl, 1e30)))
        acc_ref[4] = jnp.maximum(acc_ref[4], jnp.max(nl))
        bits_ref[pl.ds(i * rows, rows), pl.ds(h * hw, hw)] = nl

    @pl.when(i == pl.num_programs(0) - 1)
    def _select():
        pos_sum = acc_ref[0]
        neg_sum = acc_ref[1]
        pos_loss_sum = acc_ref[2]
        pos_cnt = jnp.floor(pos_sum)
        k = jnp.floor(jnp.minimum(neg_sum, pos_sum * _RATIO))

        def all_negs():
            return acc_ref[5]

        def search():
            def counts_ge(t1, t2, t3):
                def blk(j, c):
                    x = bits_ref[pl.ds(j * _CR, _CR), :]
                    xb = lax.bitcast_convert_type(x, jnp.int32)
                    return (c[0] + jnp.sum(jnp.where(xb >= t1, 1.0, 0.0)),
                            c[1] + jnp.sum(jnp.where(xb >= t2, 1.0, 0.0)),
                            c[2] + jnp.sum(jnp.where(xb >= t3, 1.0, 0.0)))
                return lax.fori_loop(0, _CHUNK, blk, (0.0, 0.0, 0.0))

            def quad(lohi):
                lo, hi = lohi
                d = (hi - lo + 3) // 4
                t1 = lo + d
                t2 = t1 + d
                t3 = t2 + d
                c1, c2, c3 = counts_ge(t1, t2, t3)
                lo = jnp.where(c3 >= k, t3,
                               jnp.where(c2 >= k, t2,
                                         jnp.where(c1 >= k, t1, lo)))
                hi = jnp.where(c1 < k, t1,
                               jnp.where(c2 < k, t2,
                                         jnp.where(c3 < k, t3, hi)))
                return (lo, hi)

            lo0 = jnp.maximum(
                lax.bitcast_convert_type(acc_ref[3], jnp.int32),
                jnp.int32(1))
            hi0 = jnp.maximum(
                lax.bitcast_convert_type(acc_ref[4], jnp.int32) + 1,
                lo0 + 1)
            v, _hi = lax.while_loop(lambda lh: lh[1] - lh[0] > 1, quad,
                                    (lo0, hi0))
            lossv = lax.bitcast_convert_type(v, jnp.float32)

            def blk2(j, carry):
                cgt, sgt = carry
                x = bits_ref[pl.ds(j * _CR, _CR), :]
                xb = lax.bitcast_convert_type(x, jnp.int32)
                gt = xb > v
                cgt += jnp.sum(jnp.where(gt, 1.0, 0.0))
                sgt += jnp.sum(jnp.where(gt, x, 0.0))
                return (cgt, sgt)

            cgt, sgt = lax.fori_loop(0, _CHUNK, blk2, (0.0, 0.0))
            return sgt + jnp.where(k > cgt, (k - cgt) * lossv, 0.0)

        top_neg = lax.cond(k >= neg_sum, all_negs, search)
        out_ref[0, 0] = (pos_loss_sum + top_neg) / (pos_cnt + k + _EPS)


def kernel(cls_score, label, mask):
    out = pl.pallas_call(
        _ohem_body,
        grid=(_GRID,),
        in_specs=[
            pl.BlockSpec((_B // _GRID, _H, _W // 2), lambda i: (i, 0, 0)),
            pl.BlockSpec((_B // _GRID, _H, _W // 2), lambda i: (i, 0, 1)),
            pl.BlockSpec((_B // _GRID, _H, _W // 2), lambda i: (i, 0, 0)),
            pl.BlockSpec((_B // _GRID, _H, _W // 2), lambda i: (i, 0, 1)),
            pl.BlockSpec((_B // _GRID, _H, _W // 2), lambda i: (i, 0, 0)),
            pl.BlockSpec((_B // _GRID, _H, _W // 2), lambda i: (i, 0, 1)),
        ],
        out_specs=pl.BlockSpec(memory_space=pltpu.SMEM),
        out_shape=jax.ShapeDtypeStruct((1, 1), jnp.float32),
        scratch_shapes=[
            pltpu.VMEM((_ROWS, _W), jnp.float32),
            pltpu.SMEM((6,), jnp.float32),
        ],
        compiler_params=pltpu.CompilerParams(
            dimension_semantics=("arbitrary",),
        ),
    )(cls_score, cls_score, label, label, mask, mask)
    return out.reshape(())
